# initial kernel scaffold (unmeasured)
import jax
import jax.numpy as jnp
from jax import lax
from jax.experimental import pallas as pl
from jax.experimental.pallas import tpu as pltpu

N_DEV = 4
SQ = 1024
SKV = 1024
DM = 1024
HQ_LOCAL = 8
DH = 128
BLK = 64
SCALE = 0.08838834764831843


def _body(x_ref, wq_ref, wo_ref, k_hbm, v_hbm, out_ref,
          comm_ref, kbuf, vbuf, send_sems, recv_sems, load_sems):
    my = lax.axis_index("i")
    right = lax.rem(my + 1, N_DEV)
    left = lax.rem(my + N_DEV - 1, N_DEV)

    barrier_sem = pltpu.get_barrier_semaphore()
    for nbr in (left, right):
        pl.semaphore_signal(
            barrier_sem, inc=1,
            device_id=(nbr,), device_id_type=pl.DeviceIdType.MESH,
        )
    pl.semaphore_wait(barrier_sem, 2)

    comm_ref[0, 0] = wq_ref[...]
    comm_ref[0, 1] = wo_ref[...]

    x_val = x_ref[...]

    rowb = lax.broadcasted_iota(jnp.int32, (SQ, SKV), 0) // BLK
    colb = lax.broadcasted_iota(jnp.int32, (SQ, SKV), 1) // BLK
    mask = colb <= rowb

    for a in range(N_DEV):
        if a > 0:
            send_slot = (a - 1) % 2
            recv_slot = a % 2
            rdma = pltpu.make_async_remote_copy(
                src_ref=comm_ref.at[send_slot],
                dst_ref=comm_ref.at[recv_slot],
                send_sem=send_sems.at[send_slot],
                recv_sem=recv_sems.at[recv_slot],
                device_id=(right,),
                device_id_type=pl.DeviceIdType.MESH,
            )
            rdma.start()
            rdma.wait()

        ck = pltpu.make_async_copy(
            k_hbm.at[pl.ds(a * HQ_LOCAL, HQ_LOCAL)], kbuf, load_sems.at[0])
        cv = pltpu.make_async_copy(
            v_hbm.at[pl.ds(a * HQ_LOCAL, HQ_LOCAL)], vbuf, load_sems.at[1])
        ck.start()
        cv.start()
        ck.wait()
        cv.wait()

        if a == 0:
            wq_c = wq_ref[...]
            wo_c = wo_ref[...]
        else:
            wq_c = comm_ref[a % 2, 0]
            wo_c = comm_ref[a % 2, 1]

        q_all = jnp.dot(x_val, wq_c, preferred_element_type=jnp.float32)
        ctxs = []
        for hl in range(HQ_LOCAL):
            q = q_all[:, hl * DH:(hl + 1) * DH] * SCALE
            s = lax.dot_general(
                q, kbuf[hl], (((1,), (1,)), ((), ())),
                preferred_element_type=jnp.float32,
            )
            s = jnp.where(mask, s, -1e9)
            m = jnp.max(s, axis=1, keepdims=True)
            w = jnp.exp(s - m)
            den = jnp.sum(w, axis=1, keepdims=True)
            w = w / den
            ctxs.append(jnp.dot(w, vbuf[hl], preferred_element_type=jnp.float32))
        part = jnp.dot(jnp.concatenate(ctxs, axis=1), wo_c,
                       preferred_element_type=jnp.float32)
        if a == 0:
            out_ref[0] = part
        else:
            out_ref[0] = out_ref[0] + part


def kernel(x, Wq, K_ext, V_ext, Wo):
    my = lax.axis_index("i")

    x2 = x[0]
    K = jnp.take(K_ext, my, axis=0)
    V = jnp.take(V_ext, my, axis=0)
    K = jnp.moveaxis(K, 1, 0)
    V = jnp.moveaxis(V, 1, 0)
    chunk_order = (my - jnp.arange(N_DEV)) % N_DEV
    head_order = (chunk_order[:, None] * HQ_LOCAL
                  + jnp.arange(HQ_LOCAL)[None, :]).reshape(-1)
    K = jnp.take(K, head_order, axis=0)
    V = jnp.take(V, head_order, axis=0)

    out = pl.pallas_call(
        _body,
        out_shape=jax.ShapeDtypeStruct((1, SQ, DM), jnp.float32),
        in_specs=[
            pl.BlockSpec(memory_space=pltpu.VMEM),
            pl.BlockSpec(memory_space=pltpu.VMEM),
            pl.BlockSpec(memory_space=pltpu.VMEM),
            pl.BlockSpec(memory_space=pltpu.ANY),
            pl.BlockSpec(memory_space=pltpu.ANY),
        ],
        out_specs=pl.BlockSpec(memory_space=pltpu.VMEM),
        scratch_shapes=[
            pltpu.VMEM((2, 2, DM, DM), jnp.float32),
            pltpu.VMEM((HQ_LOCAL, SKV, DH), jnp.float32),
            pltpu.VMEM((HQ_LOCAL, SKV, DH), jnp.float32),
            pltpu.SemaphoreType.DMA((2,)),
            pltpu.SemaphoreType.DMA((2,)),
            pltpu.SemaphoreType.DMA((2,)),
        ],
        compiler_params=pltpu.CompilerParams(collective_id=0),
    )(x2, Wq, Wo, K, V)
    return out


# baseline (device time: 460291 ns/iter reference)
import jax
import jax.numpy as jnp
from jax import lax
from jax.experimental import pallas as pl
from jax.experimental.pallas import tpu as pltpu

N_DEV = 4
SQ = 1024
SKV = 1024
DM = 1024
HQ_LOCAL = 8
DH = 128
BLK = 64
SCALE = 0.08838834764831843


def _body(x_ref, wq_ref, wo_ref, k_hbm, v_hbm, out_ref,
          comm_ref, kbuf, vbuf, send_sems, recv_sems, load_sems):
    my = lax.axis_index("i")
    right = lax.rem(my + 1, N_DEV)
    left = lax.rem(my + N_DEV - 1, N_DEV)

    barrier_sem = pltpu.get_barrier_semaphore()
    for nbr in (left, right):
        pl.semaphore_signal(
            barrier_sem, inc=1,
            device_id=(nbr,), device_id_type=pl.DeviceIdType.MESH,
        )
    pl.semaphore_wait(barrier_sem, 2)

    comm_ref[0, 0] = wq_ref[...]
    comm_ref[0, 1] = wo_ref[...]

    x_val = x_ref[...]

    rowb = lax.broadcasted_iota(jnp.int32, (SQ, SKV), 0) // BLK
    colb = lax.broadcasted_iota(jnp.int32, (SQ, SKV), 1) // BLK
    mask = colb <= rowb

    for a in range(N_DEV):
        if a > 0:
            send_slot = (a - 1) % 2
            recv_slot = a % 2
            rdma = pltpu.make_async_remote_copy(
                src_ref=comm_ref.at[send_slot],
                dst_ref=comm_ref.at[recv_slot],
                send_sem=send_sems.at[send_slot],
                recv_sem=recv_sems.at[recv_slot],
                device_id=(right,),
                device_id_type=pl.DeviceIdType.MESH,
            )
            rdma.start()
            rdma.wait()

        ck = pltpu.make_async_copy(
            k_hbm.at[pl.ds(a * HQ_LOCAL, HQ_LOCAL)], kbuf, load_sems.at[0])
        cv = pltpu.make_async_copy(
            v_hbm.at[pl.ds(a * HQ_LOCAL, HQ_LOCAL)], vbuf, load_sems.at[1])
        ck.start()
        cv.start()
        ck.wait()
        cv.wait()

        if a == 0:
            wq_c = wq_ref[...]
            wo_c = wo_ref[...]
        else:
            wq_c = comm_ref[a % 2, 0]
            wo_c = comm_ref[a % 2, 1]

        q_all = jnp.dot(x_val, wq_c, preferred_element_type=jnp.float32)
        ctxs = []
        for hl in range(HQ_LOCAL):
            q = q_all[:, hl * DH:(hl + 1) * DH] * SCALE
            s = lax.dot_general(
                q, kbuf[hl], (((1,), (1,)), ((), ())),
                preferred_element_type=jnp.float32,
            )
            s = jnp.where(mask, s, -1e9)
            m = jnp.max(s, axis=1, keepdims=True)
            w = jnp.exp(s - m)
            den = jnp.sum(w, axis=1, keepdims=True)
            w = w / den
            ctxs.append(jnp.dot(w, vbuf[hl], preferred_element_type=jnp.float32))
        part = jnp.dot(jnp.concatenate(ctxs, axis=1), wo_c,
                       preferred_element_type=jnp.float32)
        if a == 0:
            out_ref[0] = part
        else:
            out_ref[0] = out_ref[0] + part


def kernel(x, Wq, K_ext, V_ext, Wo):
    my = lax.axis_index("i")

    x2 = x[0]
    K = jnp.take(K_ext, my, axis=0)
    V = jnp.take(V_ext, my, axis=0)
    K = jnp.moveaxis(K, 1, 0)
    V = jnp.moveaxis(V, 1, 0)
    chunk_order = (my - jnp.arange(N_DEV)) % N_DEV
    head_order = (chunk_order[:, None] * HQ_LOCAL
                  + jnp.arange(HQ_LOCAL)[None, :]).reshape(-1)
    K = jnp.take(K, head_order, axis=0)
    V = jnp.take(V, head_order, axis=0)

    out = pl.pallas_call(
        _body,
        out_shape=jax.ShapeDtypeStruct((1, SQ, DM), jnp.float32),
        in_specs=[
            pl.BlockSpec(memory_space=pltpu.VMEM),
            pl.BlockSpec(memory_space=pltpu.VMEM),
            pl.BlockSpec(memory_space=pltpu.VMEM),
            pl.BlockSpec(memory_space=pl.ANY),
            pl.BlockSpec(memory_space=pl.ANY),
        ],
        out_specs=pl.BlockSpec(memory_space=pltpu.VMEM),
        scratch_shapes=[
            pltpu.VMEM((2, 2, DM, DM), jnp.float32),
            pltpu.VMEM((HQ_LOCAL, SKV, DH), jnp.float32),
            pltpu.VMEM((HQ_LOCAL, SKV, DH), jnp.float32),
            pltpu.SemaphoreType.DMA((2,)),
            pltpu.SemaphoreType.DMA((2,)),
            pltpu.SemaphoreType.DMA((2,)),
        ],
        compiler_params=pltpu.CompilerParams(
            collective_id=0,
            vmem_limit_bytes=100 * 1024 * 1024,
        ),
    )(x2, Wq, Wo, K, V)
    return out


# device time: 231825 ns/iter; 1.9855x vs baseline; 1.9855x over previous
import jax
import jax.numpy as jnp
from jax import lax
from jax.experimental import pallas as pl
from jax.experimental.pallas import tpu as pltpu

N_DEV = 4
SQ = 1024
SKV = 1024
DM = 1024
HQ_LOCAL = 8
DH = 128
BLK = 64
SCALE = 0.08838834764831843


def _body(x_ref, wq_ref, wo_ref, k_hbm, v_hbm, out_ref,
          comm_ref, kbuf, vbuf, q_buf, ctx_buf, bias_ref,
          send_sems, recv_sems, load_sems):
    my = lax.axis_index("i")
    right = lax.rem(my + 1, N_DEV)
    left = lax.rem(my + N_DEV - 1, N_DEV)

    barrier_sem = pltpu.get_barrier_semaphore()
    for nbr in (left, right):
        pl.semaphore_signal(
            barrier_sem, inc=1,
            device_id=(nbr,), device_id_type=pl.DeviceIdType.MESH,
        )
    pl.semaphore_wait(barrier_sem, 2)

    r1a = pltpu.make_async_remote_copy(
        src_ref=wq_ref, dst_ref=comm_ref.at[0, 0],
        send_sem=send_sems.at[0], recv_sem=recv_sems.at[0],
        device_id=(right,), device_id_type=pl.DeviceIdType.MESH,
    )
    r1b = pltpu.make_async_remote_copy(
        src_ref=wo_ref, dst_ref=comm_ref.at[0, 1],
        send_sem=send_sems.at[1], recv_sem=recv_sems.at[1],
        device_id=(right,), device_id_type=pl.DeviceIdType.MESH,
    )
    r2 = pltpu.make_async_remote_copy(
        src_ref=comm_ref.at[0], dst_ref=comm_ref.at[1],
        send_sem=send_sems.at[2], recv_sem=recv_sems.at[2],
        device_id=(right,), device_id_type=pl.DeviceIdType.MESH,
    )
    r3 = pltpu.make_async_remote_copy(
        src_ref=comm_ref.at[1], dst_ref=comm_ref.at[2],
        send_sem=send_sems.at[3], recv_sem=recv_sems.at[3],
        device_id=(right,), device_id_type=pl.DeviceIdType.MESH,
    )
    r1a.start()
    r1b.start()

    def kv_load(a):
        slot = a % 2
        ck = pltpu.make_async_copy(
            k_hbm.at[pl.ds(a * HQ_LOCAL, HQ_LOCAL)], kbuf.at[slot],
            load_sems.at[2 * slot])
        cv = pltpu.make_async_copy(
            v_hbm.at[pl.ds(a * HQ_LOCAL, HQ_LOCAL)], vbuf.at[slot],
            load_sems.at[2 * slot + 1])
        return ck, cv

    loads = [kv_load(a) for a in range(N_DEV)]
    for d in loads[0]:
        d.start()

    rowb = lax.broadcasted_iota(jnp.int32, (SQ, SKV), 0) // BLK
    colb = lax.broadcasted_iota(jnp.int32, (SQ, SKV), 1) // BLK
    bias_ref[...] = jnp.where(colb <= rowb, 0.0, -1e9).astype(jnp.float32)

    for a in range(N_DEV):
        if a == 1:
            r1a.wait_recv()
            r1b.wait_recv()
            r2.start()
        elif a == 2:
            r2.wait_recv()
            r3.start()
        elif a == 3:
            r3.wait_recv()

        for d in loads[a]:
            d.wait()
        if a + 1 < N_DEV:
            for d in loads[a + 1]:
                d.start()

        if a == 0:
            wq_c = wq_ref[...]
            wo_c = wo_ref[...]
        else:
            wq_c = comm_ref[a - 1, 0]
            wo_c = comm_ref[a - 1, 1]

        slot = a % 2
        q_all = jnp.dot(x_ref[...], wq_c, preferred_element_type=jnp.float32)
        q_buf[...] = q_all.astype(jnp.bfloat16)

        def head(hl, carry):
            q = q_buf[:, pl.ds(hl * DH, DH)]
            s = lax.dot_general(
                q, kbuf[slot, hl], (((1,), (1,)), ((), ())),
                preferred_element_type=jnp.float32,
            )
            s = s + bias_ref[...]
            m = jnp.max(s, axis=1, keepdims=True)
            w = jnp.exp(s - m)
            den = jnp.sum(w, axis=1, keepdims=True)
            ctx = jnp.dot(w.astype(jnp.bfloat16), vbuf[slot, hl],
                          preferred_element_type=jnp.float32)
            ctx = ctx * (1.0 / den)
            ctx_buf[:, pl.ds(hl * DH, DH)] = ctx.astype(jnp.bfloat16)
            return carry

        lax.fori_loop(0, HQ_LOCAL, head, 0)

        part = jnp.dot(ctx_buf[...], wo_c, preferred_element_type=jnp.float32)
        if a == 0:
            out_ref[0] = part
        else:
            out_ref[0] = out_ref[0] + part

    r1a.wait_send()
    r1b.wait_send()
    r2.wait_send()
    r3.wait_send()


def kernel(x, Wq, K_ext, V_ext, Wo):
    my = lax.axis_index("i")

    x2 = (x[0] * SCALE).astype(jnp.bfloat16)
    K = jnp.take(K_ext, my, axis=0)
    V = jnp.take(V_ext, my, axis=0)
    K = jnp.moveaxis(K, 1, 0)
    V = jnp.moveaxis(V, 1, 0)
    chunk_order = (my - jnp.arange(N_DEV)) % N_DEV
    head_order = (chunk_order[:, None] * HQ_LOCAL
                  + jnp.arange(HQ_LOCAL)[None, :]).reshape(-1)
    K = jnp.take(K, head_order, axis=0).astype(jnp.bfloat16)
    V = jnp.take(V, head_order, axis=0).astype(jnp.bfloat16)

    out = pl.pallas_call(
        _body,
        out_shape=jax.ShapeDtypeStruct((1, SQ, DM), jnp.float32),
        in_specs=[
            pl.BlockSpec(memory_space=pltpu.VMEM),
            pl.BlockSpec(memory_space=pltpu.VMEM),
            pl.BlockSpec(memory_space=pltpu.VMEM),
            pl.BlockSpec(memory_space=pl.ANY),
            pl.BlockSpec(memory_space=pl.ANY),
        ],
        out_specs=pl.BlockSpec(memory_space=pltpu.VMEM),
        scratch_shapes=[
            pltpu.VMEM((3, 2, DM, DM), jnp.bfloat16),
            pltpu.VMEM((2, HQ_LOCAL, SKV, DH), jnp.bfloat16),
            pltpu.VMEM((2, HQ_LOCAL, SKV, DH), jnp.bfloat16),
            pltpu.VMEM((SQ, DM), jnp.bfloat16),
            pltpu.VMEM((SQ, DM), jnp.bfloat16),
            pltpu.VMEM((SQ, SKV), jnp.float32),
            pltpu.SemaphoreType.DMA((4,)),
            pltpu.SemaphoreType.DMA((4,)),
            pltpu.SemaphoreType.DMA((4,)),
        ],
        compiler_params=pltpu.CompilerParams(
            collective_id=0,
            vmem_limit_bytes=100 * 1024 * 1024,
        ),
    )(x2, Wq.astype(jnp.bfloat16), Wo.astype(jnp.bfloat16), K, V)
    return out


# device time: 158291 ns/iter; 2.9079x vs baseline; 1.4645x over previous
import jax
import jax.numpy as jnp
from jax import lax
from jax.experimental import pallas as pl
from jax.experimental.pallas import tpu as pltpu

N_DEV = 4
SQ = 1024
SKV = 1024
DM = 1024
HQ_LOCAL = 8
DH = 128
BLK = 64
QBAND = 256
N_BANDS = SQ // QBAND
HALF = DM // 2
SCALE = 0.08838834764831843


def _body(x_ref, wq_ref, wo_ref, k_hbm, v_hbm, out_ref,
          comm_ref, kbuf, vbuf, q_buf, ctx_buf, bias_ref,
          send_sems, recv_sems, load_sems):
    my = lax.axis_index("i")
    right = lax.rem(my + 1, N_DEV)
    left = lax.rem(my + N_DEV - 1, N_DEV)

    barrier_sem = pltpu.get_barrier_semaphore()
    for nbr in (left, right):
        pl.semaphore_signal(
            barrier_sem, inc=1,
            device_id=(nbr,), device_id_type=pl.DeviceIdType.MESH,
        )
    pl.semaphore_wait(barrier_sem, 2)

    def remote(src, dst, s, dev):
        return pltpu.make_async_remote_copy(
            src_ref=src, dst_ref=dst,
            send_sem=send_sems.at[s], recv_sem=recv_sems.at[s],
            device_id=(dev,), device_id_type=pl.DeviceIdType.MESH,
        )

    a1 = remote(wq_ref, comm_ref.at[0, 0], 0, right)
    a2 = remote(wo_ref, comm_ref.at[0, 1], 1, right)
    a3 = remote(wq_ref, comm_ref.at[1, 0], 2, left)
    a4 = remote(wo_ref, comm_ref.at[1, 1], 3, left)
    b1 = remote(comm_ref.at[0, 0, pl.ds(0, HALF)],
                comm_ref.at[2, 0, pl.ds(0, HALF)], 4, right)
    b2 = remote(comm_ref.at[0, 1, pl.ds(0, HALF)],
                comm_ref.at[2, 1, pl.ds(0, HALF)], 5, right)
    b3 = remote(comm_ref.at[1, 0, pl.ds(HALF, HALF)],
                comm_ref.at[2, 0, pl.ds(HALF, HALF)], 6, left)
    b4 = remote(comm_ref.at[1, 1, pl.ds(HALF, HALF)],
                comm_ref.at[2, 1, pl.ds(HALF, HALF)], 7, left)

    a1.start()
    a2.start()
    a3.start()
    a4.start()

    chunk_ids = [my,
                 lax.rem(my + N_DEV - 1, N_DEV),
                 lax.rem(my + 1, N_DEV),
                 lax.rem(my + 2, N_DEV)]

    def kv_load(a):
        slot = a % 2
        off = chunk_ids[a] * HQ_LOCAL
        ck = pltpu.make_async_copy(
            k_hbm.at[pl.ds(off, HQ_LOCAL)], kbuf.at[slot],
            load_sems.at[2 * slot])
        cv = pltpu.make_async_copy(
            v_hbm.at[pl.ds(off, HQ_LOCAL)], vbuf.at[slot],
            load_sems.at[2 * slot + 1])
        return ck, cv

    loads = [kv_load(a) for a in range(N_DEV)]
    for d in loads[0]:
        d.start()

    rowb = lax.broadcasted_iota(jnp.int32, (SQ, SKV), 0) // BLK
    colb = lax.broadcasted_iota(jnp.int32, (SQ, SKV), 1) // BLK
    bias_ref[...] = jnp.where(colb <= rowb, 0.0, -1e9).astype(jnp.float32)

    for a in range(N_DEV):
        if a == 1:
            a1.wait_recv()
            a2.wait_recv()
            b1.start()
            b2.start()
            a3.wait_recv()
            a4.wait_recv()
            b3.start()
            b4.start()
        elif a == 3:
            b1.wait_recv()
            b2.wait_recv()
            b3.wait_recv()
            b4.wait_recv()

        for d in loads[a]:
            d.wait()
        if a + 1 < N_DEV:
            for d in loads[a + 1]:
                d.start()

        if a == 0:
            wq_c = wq_ref[...]
            wo_c = wo_ref[...]
        else:
            wq_c = comm_ref[a - 1, 0]
            wo_c = comm_ref[a - 1, 1]

        slot = a % 2
        q_all = jnp.dot(x_ref[...], wq_c, preferred_element_type=jnp.float32)
        q_buf[...] = q_all.astype(jnp.bfloat16)

        def head(hl, carry):
            for b in range(N_BANDS):
                cols = (b + 1) * QBAND
                q = q_buf[pl.ds(b * QBAND, QBAND), pl.ds(hl * DH, DH)]
                s = lax.dot_general(
                    q, kbuf[slot, hl, pl.ds(0, cols)],
                    (((1,), (1,)), ((), ())),
                    preferred_element_type=jnp.float32,
                )
                s = s + bias_ref[pl.ds(b * QBAND, QBAND), pl.ds(0, cols)]
                m = jnp.max(s, axis=1, keepdims=True)
                w = jnp.exp(s - m)
                den = jnp.sum(w, axis=1, keepdims=True)
                ctx = jnp.dot(w.astype(jnp.bfloat16),
                              vbuf[slot, hl, pl.ds(0, cols)],
                              preferred_element_type=jnp.float32)
                ctx = ctx * (1.0 / den)
                ctx_buf[pl.ds(b * QBAND, QBAND), pl.ds(hl * DH, DH)] = (
                    ctx.astype(jnp.bfloat16))
            return carry

        lax.fori_loop(0, HQ_LOCAL, head, 0)

        part = jnp.dot(ctx_buf[...], wo_c, preferred_element_type=jnp.float32)
        if a == 0:
            out_ref[0] = part
        else:
            out_ref[0] = out_ref[0] + part

    for r in (a1, a2, a3, a4, b1, b2, b3, b4):
        r.wait_send()


def kernel(x, Wq, K_ext, V_ext, Wo):
    my = lax.axis_index("i")

    x2 = (x[0] * SCALE).astype(jnp.bfloat16)
    K = jnp.take(K_ext, my, axis=0)
    V = jnp.take(V_ext, my, axis=0)
    K = jnp.moveaxis(K, 1, 0).astype(jnp.bfloat16)
    V = jnp.moveaxis(V, 1, 0).astype(jnp.bfloat16)

    out = pl.pallas_call(
        _body,
        out_shape=jax.ShapeDtypeStruct((1, SQ, DM), jnp.float32),
        in_specs=[
            pl.BlockSpec(memory_space=pltpu.VMEM),
            pl.BlockSpec(memory_space=pltpu.VMEM),
            pl.BlockSpec(memory_space=pltpu.VMEM),
            pl.BlockSpec(memory_space=pl.ANY),
            pl.BlockSpec(memory_space=pl.ANY),
        ],
        out_specs=pl.BlockSpec(memory_space=pltpu.VMEM),
        scratch_shapes=[
            pltpu.VMEM((3, 2, DM, DM), jnp.bfloat16),
            pltpu.VMEM((2, HQ_LOCAL, SKV, DH), jnp.bfloat16),
            pltpu.VMEM((2, HQ_LOCAL, SKV, DH), jnp.bfloat16),
            pltpu.VMEM((SQ, DM), jnp.bfloat16),
            pltpu.VMEM((SQ, DM), jnp.bfloat16),
            pltpu.VMEM((SQ, SKV), jnp.float32),
            pltpu.SemaphoreType.DMA((8,)),
            pltpu.SemaphoreType.DMA((8,)),
            pltpu.SemaphoreType.DMA((4,)),
        ],
        compiler_params=pltpu.CompilerParams(
            collective_id=0,
            vmem_limit_bytes=100 * 1024 * 1024,
        ),
    )(x2, Wq.astype(jnp.bfloat16), Wo.astype(jnp.bfloat16), K, V)
    return out


# device time: 130028 ns/iter; 3.5399x vs baseline; 1.2174x over previous
import jax
import jax.numpy as jnp
from jax import lax
from jax.experimental import pallas as pl
from jax.experimental.pallas import tpu as pltpu

N_DEV = 4
SQ = 1024
SKV = 1024
DM = 1024
HQ_LOCAL = 8
DH = 128
BLK = 64
QBAND = 256
N_BANDS = SQ // QBAND
HALF = DM // 2
SCALE = 0.08838834764831843


def _body(x_ref, wq_ref, wo_ref, k_hbm, v_hbm, out_ref,
          comm_ref, kbuf, vbuf, q_buf, ctx_buf, bias_ref,
          send_sems, recv_sems, load_sems):
    my = lax.axis_index("i")
    right = lax.rem(my + 1, N_DEV)
    left = lax.rem(my + N_DEV - 1, N_DEV)

    barrier_sem = pltpu.get_barrier_semaphore()
    for nbr in (left, right):
        pl.semaphore_signal(
            barrier_sem, inc=1,
            device_id=(nbr,), device_id_type=pl.DeviceIdType.MESH,
        )
    pl.semaphore_wait(barrier_sem, 2)

    def remote(src, dst, s, dev):
        return pltpu.make_async_remote_copy(
            src_ref=src, dst_ref=dst,
            send_sem=send_sems.at[s], recv_sem=recv_sems.at[s],
            device_id=(dev,), device_id_type=pl.DeviceIdType.MESH,
        )

    a1 = remote(wq_ref, comm_ref.at[0, 0], 0, right)
    a2 = remote(wo_ref, comm_ref.at[0, 1], 1, right)
    a3 = remote(wq_ref, comm_ref.at[1, 0], 2, left)
    a4 = remote(wo_ref, comm_ref.at[1, 1], 3, left)
    b1 = remote(comm_ref.at[0, 0, pl.ds(0, HALF)],
                comm_ref.at[2, 0, pl.ds(0, HALF)], 4, right)
    b2 = remote(comm_ref.at[0, 1, pl.ds(0, HALF)],
                comm_ref.at[2, 1, pl.ds(0, HALF)], 5, right)
    b3 = remote(comm_ref.at[1, 0, pl.ds(HALF, HALF)],
                comm_ref.at[2, 0, pl.ds(HALF, HALF)], 6, left)
    b4 = remote(comm_ref.at[1, 1, pl.ds(HALF, HALF)],
                comm_ref.at[2, 1, pl.ds(HALF, HALF)], 7, left)

    a1.start()
    a2.start()
    a3.start()
    a4.start()

    chunk_ids = [my,
                 lax.rem(my + N_DEV - 1, N_DEV),
                 lax.rem(my + 1, N_DEV),
                 lax.rem(my + 2, N_DEV)]

    def kv_load(a):
        slot = a % 2
        cps = []
        for hl in range(HQ_LOCAL):
            h = chunk_ids[a] * HQ_LOCAL + hl
            cps.append(pltpu.make_async_copy(
                k_hbm.at[my, :, h], kbuf.at[slot, hl],
                load_sems.at[slot, 0, hl]))
            cps.append(pltpu.make_async_copy(
                v_hbm.at[my, :, h], vbuf.at[slot, hl],
                load_sems.at[slot, 1, hl]))
        return cps

    loads = [kv_load(a) for a in range(N_DEV)]
    for d in loads[0]:
        d.start()

    rowb = lax.broadcasted_iota(jnp.int32, (SQ, SKV), 0) // BLK
    colb = lax.broadcasted_iota(jnp.int32, (SQ, SKV), 1) // BLK
    bias_ref[...] = jnp.where(colb <= rowb, 0.0, -1e9).astype(jnp.float32)

    for a in range(N_DEV):
        if a == 1:
            a1.wait_recv()
            a2.wait_recv()
            b1.start()
            b2.start()
            a3.wait_recv()
            a4.wait_recv()
            b3.start()
            b4.start()
        elif a == 3:
            b1.wait_recv()
            b2.wait_recv()
            b3.wait_recv()
            b4.wait_recv()

        for d in loads[a]:
            d.wait()
        if a + 1 < N_DEV:
            for d in loads[a + 1]:
                d.start()

        if a == 0:
            wq_c = wq_ref[...]
            wo_c = wo_ref[...]
        else:
            wq_c = comm_ref[a - 1, 0]
            wo_c = comm_ref[a - 1, 1]

        slot = a % 2
        q_all = jnp.dot(x_ref[...], wq_c, preferred_element_type=jnp.float32)
        q_buf[...] = q_all.astype(jnp.bfloat16)

        def head(hl, carry):
            for b in range(N_BANDS):
                cols = (b + 1) * QBAND
                q = q_buf[pl.ds(b * QBAND, QBAND), pl.ds(hl * DH, DH)]
                s = lax.dot_general(
                    q, kbuf[slot, hl, pl.ds(0, cols)].astype(jnp.bfloat16),
                    (((1,), (1,)), ((), ())),
                    preferred_element_type=jnp.float32,
                )
                s = s + bias_ref[pl.ds(b * QBAND, QBAND), pl.ds(0, cols)]
                m = jnp.max(s, axis=1, keepdims=True)
                w = jnp.exp(s - m)
                den = jnp.sum(w, axis=1, keepdims=True)
                ctx = jnp.dot(w.astype(jnp.bfloat16),
                              vbuf[slot, hl, pl.ds(0, cols)].astype(
                                  jnp.bfloat16),
                              preferred_element_type=jnp.float32)
                ctx = ctx * (1.0 / den)
                ctx_buf[pl.ds(b * QBAND, QBAND), pl.ds(hl * DH, DH)] = (
                    ctx.astype(jnp.bfloat16))
            return carry

        lax.fori_loop(0, HQ_LOCAL, head, 0)

        part = jnp.dot(ctx_buf[...], wo_c, preferred_element_type=jnp.float32)
        if a == 0:
            out_ref[0] = part
        else:
            out_ref[0] = out_ref[0] + part

    for r in (a1, a2, a3, a4, b1, b2, b3, b4):
        r.wait_send()


def kernel(x, Wq, K_ext, V_ext, Wo):
    my = lax.axis_index("i")

    x2 = (x[0] * SCALE).astype(jnp.bfloat16)

    out = pl.pallas_call(
        _body,
        out_shape=jax.ShapeDtypeStruct((1, SQ, DM), jnp.float32),
        in_specs=[
            pl.BlockSpec(memory_space=pltpu.VMEM),
            pl.BlockSpec(memory_space=pltpu.VMEM),
            pl.BlockSpec(memory_space=pltpu.VMEM),
            pl.BlockSpec(memory_space=pl.ANY),
            pl.BlockSpec(memory_space=pl.ANY),
        ],
        out_specs=pl.BlockSpec(memory_space=pltpu.VMEM),
        scratch_shapes=[
            pltpu.VMEM((3, 2, DM, DM), jnp.bfloat16),
            pltpu.VMEM((2, HQ_LOCAL, SKV, DH), jnp.float32),
            pltpu.VMEM((2, HQ_LOCAL, SKV, DH), jnp.float32),
            pltpu.VMEM((SQ, DM), jnp.bfloat16),
            pltpu.VMEM((SQ, DM), jnp.bfloat16),
            pltpu.VMEM((SQ, SKV), jnp.float32),
            pltpu.SemaphoreType.DMA((8,)),
            pltpu.SemaphoreType.DMA((8,)),
            pltpu.SemaphoreType.DMA((2, 2, HQ_LOCAL)),
        ],
        compiler_params=pltpu.CompilerParams(
            collective_id=0,
            vmem_limit_bytes=100 * 1024 * 1024,
        ),
    )(x2, Wq.astype(jnp.bfloat16), Wo.astype(jnp.bfloat16), K_ext, V_ext)
    return out


# device time: 119862 ns/iter; 3.8402x vs baseline; 1.0848x over previous
import jax
import jax.numpy as jnp
from jax import lax
from jax.experimental import pallas as pl
from jax.experimental.pallas import tpu as pltpu

N_DEV = 4
SQ = 1024
SKV = 1024
DM = 1024
HQ_LOCAL = 8
DH = 128
BLK = 64
QBAND = 256
N_BANDS = SQ // QBAND
HALF = DM // 2
SCALE = 0.08838834764831843


def _body(x_ref, wq_ref, wo_ref, k_hbm, v_hbm, out_ref,
          comm_ref, kbuf, vbuf, q_buf, ctx_buf, bias_ref,
          send_sems, recv_sems, load_sems):
    my = lax.axis_index("i")
    right = lax.rem(my + 1, N_DEV)
    left = lax.rem(my + N_DEV - 1, N_DEV)

    barrier_sem = pltpu.get_barrier_semaphore()
    for nbr in (left, right):
        pl.semaphore_signal(
            barrier_sem, inc=1,
            device_id=(nbr,), device_id_type=pl.DeviceIdType.MESH,
        )
    pl.semaphore_wait(barrier_sem, 2)

    def remote(src, dst, s, dev):
        return pltpu.make_async_remote_copy(
            src_ref=src, dst_ref=dst,
            send_sem=send_sems.at[s], recv_sem=recv_sems.at[s],
            device_id=(dev,), device_id_type=pl.DeviceIdType.MESH,
        )

    a_sends = []
    for d, dev in ((0, right), (1, left)):
        for g in range(2):
            cs = pl.ds(g * HALF, HALF)
            base = 4 * d + 2 * g
            a_sends.append(remote(wq_ref.at[:, cs],
                                  comm_ref.at[d, 0, :, cs], base, dev))
            a_sends.append(remote(wo_ref.at[cs],
                                  comm_ref.at[d, 1, cs], base + 1, dev))
    b1 = remote(comm_ref.at[0, 0, pl.ds(0, HALF)],
                comm_ref.at[2, 0, pl.ds(0, HALF)], 8, right)
    b2 = remote(comm_ref.at[0, 1, pl.ds(0, HALF)],
                comm_ref.at[2, 1, pl.ds(0, HALF)], 9, right)
    b3 = remote(comm_ref.at[1, 0, pl.ds(HALF, HALF)],
                comm_ref.at[2, 0, pl.ds(HALF, HALF)], 10, left)
    b4 = remote(comm_ref.at[1, 1, pl.ds(HALF, HALF)],
                comm_ref.at[2, 1, pl.ds(HALF, HALF)], 11, left)

    for r in a_sends:
        r.start()

    chunk_ids = [my,
                 lax.rem(my + N_DEV - 1, N_DEV),
                 lax.rem(my + 1, N_DEV),
                 lax.rem(my + 2, N_DEV)]

    def kv_load(a):
        slot = a % 2
        cps = []
        for hl in range(HQ_LOCAL):
            h = chunk_ids[a] * HQ_LOCAL + hl
            cps.append(pltpu.make_async_copy(
                k_hbm.at[my, :, h], kbuf.at[slot, hl],
                load_sems.at[slot, 0, hl]))
            cps.append(pltpu.make_async_copy(
                v_hbm.at[my, :, h], vbuf.at[slot, hl],
                load_sems.at[slot, 1, hl]))
        return cps

    loads = [kv_load(a) for a in range(N_DEV)]
    for d in loads[0]:
        d.start()

    rowb = lax.broadcasted_iota(jnp.int32, (SQ, SKV), 0) // BLK
    colb = lax.broadcasted_iota(jnp.int32, (SQ, SKV), 1) // BLK
    bias_ref[...] = jnp.where(colb <= rowb, 0.0, -1e9).astype(jnp.float32)

    x_bf = x_ref[...]

    def process(slot, wq_c, wo_c, col0, ncols, first):
        qg = jnp.dot(x_bf, wq_c, preferred_element_type=jnp.float32)
        q_buf[:, pl.ds(col0, ncols)] = qg.astype(jnp.bfloat16)

        def head(i, carry):
            hl = col0 // DH + i
            for b in range(N_BANDS):
                cols = (b + 1) * QBAND
                q = q_buf[pl.ds(b * QBAND, QBAND), pl.ds(hl * DH, DH)]
                s = lax.dot_general(
                    q, kbuf[slot, hl, pl.ds(0, cols)].astype(jnp.bfloat16),
                    (((1,), (1,)), ((), ())),
                    preferred_element_type=jnp.float32,
                )
                s = s + bias_ref[pl.ds(b * QBAND, QBAND), pl.ds(0, cols)]
                m = jnp.max(s, axis=1, keepdims=True)
                w = jnp.exp(s - m)
                den = jnp.sum(w, axis=1, keepdims=True)
                ctx = jnp.dot(w.astype(jnp.bfloat16),
                              vbuf[slot, hl, pl.ds(0, cols)].astype(
                                  jnp.bfloat16),
                              preferred_element_type=jnp.float32)
                ctx = ctx * (1.0 / den)
                ctx_buf[pl.ds(b * QBAND, QBAND), pl.ds(hl * DH, DH)] = (
                    ctx.astype(jnp.bfloat16))
            return carry

        lax.fori_loop(0, ncols // DH, head, 0)

        part = jnp.dot(ctx_buf[:, pl.ds(col0, ncols)], wo_c,
                       preferred_element_type=jnp.float32)
        if first:
            out_ref[0] = part
        else:
            out_ref[0] = out_ref[0] + part

    for d in loads[0]:
        d.wait()
    for d in loads[1]:
        d.start()
    process(0, wq_ref[...], wo_ref[...], 0, DM, first=True)

    for d in loads[1]:
        d.wait()
    for d in loads[2]:
        d.start()
    a_sends[0].wait_recv()
    a_sends[1].wait_recv()
    process(1, comm_ref[0, 0, :, pl.ds(0, HALF)],
            comm_ref[0, 1, pl.ds(0, HALF)], 0, HALF, first=False)
    a_sends[2].wait_recv()
    a_sends[3].wait_recv()
    b1.start()
    b2.start()
    for k in (4, 5, 6, 7):
        a_sends[k].wait_recv()
    b3.start()
    b4.start()
    process(1, comm_ref[0, 0, :, pl.ds(HALF, HALF)],
            comm_ref[0, 1, pl.ds(HALF, HALF)], HALF, HALF, first=False)

    for d in loads[2]:
        d.wait()
    for d in loads[3]:
        d.start()
    process(0, comm_ref[1, 0], comm_ref[1, 1], 0, DM, first=False)

    b1.wait_recv()
    b2.wait_recv()
    b3.wait_recv()
    b4.wait_recv()
    for d in loads[3]:
        d.wait()
    process(1, comm_ref[2, 0], comm_ref[2, 1], 0, DM, first=False)

    for r in a_sends + [b1, b2, b3, b4]:
        r.wait_send()


def kernel(x, Wq, K_ext, V_ext, Wo):
    my = lax.axis_index("i")

    x2 = (x[0] * SCALE).astype(jnp.bfloat16)

    out = pl.pallas_call(
        _body,
        out_shape=jax.ShapeDtypeStruct((1, SQ, DM), jnp.float32),
        in_specs=[
            pl.BlockSpec(memory_space=pltpu.VMEM),
            pl.BlockSpec(memory_space=pltpu.VMEM),
            pl.BlockSpec(memory_space=pltpu.VMEM),
            pl.BlockSpec(memory_space=pl.ANY),
            pl.BlockSpec(memory_space=pl.ANY),
        ],
        out_specs=pl.BlockSpec(memory_space=pltpu.VMEM),
        scratch_shapes=[
            pltpu.VMEM((3, 2, DM, DM), jnp.bfloat16),
            pltpu.VMEM((2, HQ_LOCAL, SKV, DH), jnp.float32),
            pltpu.VMEM((2, HQ_LOCAL, SKV, DH), jnp.float32),
            pltpu.VMEM((SQ, DM), jnp.bfloat16),
            pltpu.VMEM((SQ, DM), jnp.bfloat16),
            pltpu.VMEM((SQ, SKV), jnp.float32),
            pltpu.SemaphoreType.DMA((12,)),
            pltpu.SemaphoreType.DMA((12,)),
            pltpu.SemaphoreType.DMA((2, 2, HQ_LOCAL)),
        ],
        compiler_params=pltpu.CompilerParams(
            collective_id=0,
            vmem_limit_bytes=100 * 1024 * 1024,
        ),
    )(x2, Wq.astype(jnp.bfloat16), Wo.astype(jnp.bfloat16), K_ext, V_ext)
    return out


# device time: 117216 ns/iter; 3.9269x vs baseline; 1.0226x over previous
import jax
import jax.numpy as jnp
from jax import lax
from jax.experimental import pallas as pl
from jax.experimental.pallas import tpu as pltpu

N_DEV = 4
SQ = 1024
SKV = 1024
DM = 1024
HQ_LOCAL = 8
DH = 128
BLK = 64
QBAND = 256
N_BANDS = SQ // QBAND
HALF = DM // 2
SCALE = 0.08838834764831843


def _body(x_ref, wq_ref, wo_ref, k_hbm, v_hbm, out_ref,
          comm_ref, kbuf, vbuf, q_buf, ctx_buf, bias_ref,
          send_sems, recv_sems, load_sems):
    my = lax.axis_index("i")
    right = lax.rem(my + 1, N_DEV)
    left = lax.rem(my + N_DEV - 1, N_DEV)

    barrier_sem = pltpu.get_barrier_semaphore()
    for nbr in (left, right):
        pl.semaphore_signal(
            barrier_sem, inc=1,
            device_id=(nbr,), device_id_type=pl.DeviceIdType.MESH,
        )
    pl.semaphore_wait(barrier_sem, 2)

    def remote(src, dst, s, dev):
        return pltpu.make_async_remote_copy(
            src_ref=src, dst_ref=dst,
            send_sem=send_sems.at[s], recv_sem=recv_sems.at[s],
            device_id=(dev,), device_id_type=pl.DeviceIdType.MESH,
        )

    a_sends = []
    for d, dev in ((0, right), (1, left)):
        for g in range(2):
            cs = pl.ds(g * HALF, HALF)
            base = 4 * d + 2 * g
            a_sends.append(remote(wq_ref.at[:, cs],
                                  comm_ref.at[d, 0, :, cs], base, dev))
            a_sends.append(remote(wo_ref.at[cs],
                                  comm_ref.at[d, 1, cs], base + 1, dev))
    b1 = remote(comm_ref.at[0, 0, pl.ds(0, HALF)],
                comm_ref.at[2, 0, pl.ds(0, HALF)], 8, right)
    b2 = remote(comm_ref.at[0, 1, pl.ds(0, HALF)],
                comm_ref.at[2, 1, pl.ds(0, HALF)], 9, right)
    b3 = remote(comm_ref.at[1, 0, pl.ds(HALF, HALF)],
                comm_ref.at[2, 0, pl.ds(HALF, HALF)], 10, left)
    b4 = remote(comm_ref.at[1, 1, pl.ds(HALF, HALF)],
                comm_ref.at[2, 1, pl.ds(HALF, HALF)], 11, left)

    for r in a_sends:
        r.start()

    chunk_ids = [my,
                 lax.rem(my + N_DEV - 1, N_DEV),
                 lax.rem(my + 1, N_DEV),
                 lax.rem(my + 2, N_DEV)]

    def kv_load(a):
        slot = a % 2
        cps = []
        for hl in range(HQ_LOCAL):
            h = chunk_ids[a] * HQ_LOCAL + hl
            cps.append(pltpu.make_async_copy(
                k_hbm.at[my, :, h], kbuf.at[slot, hl],
                load_sems.at[slot, 0, hl]))
            cps.append(pltpu.make_async_copy(
                v_hbm.at[my, :, h], vbuf.at[slot, hl],
                load_sems.at[slot, 1, hl]))
        return cps

    loads = [kv_load(a) for a in range(N_DEV)]
    for d in loads[0]:
        d.start()

    rowb = lax.broadcasted_iota(jnp.int32, (SQ, SKV), 0) // BLK
    colb = lax.broadcasted_iota(jnp.int32, (SQ, SKV), 1) // BLK
    bias_ref[...] = jnp.where(colb <= rowb, 0.0, -1e9).astype(jnp.bfloat16)

    x_bf = x_ref[...]

    def process(slot, wq_c, wo_c, col0, ncols, first):
        qg = jnp.dot(x_bf, wq_c, preferred_element_type=jnp.float32)
        q_buf[:, pl.ds(col0, ncols)] = qg.astype(jnp.bfloat16)

        def head(i, carry):
            hl = col0 // DH + i
            for b in range(N_BANDS):
                cols = (b + 1) * QBAND
                q = q_buf[pl.ds(b * QBAND, QBAND), pl.ds(hl * DH, DH)]
                s = lax.dot_general(
                    q, kbuf[slot, hl, pl.ds(0, cols)].astype(jnp.bfloat16),
                    (((1,), (1,)), ((), ())),
                    preferred_element_type=jnp.float32,
                ).astype(jnp.bfloat16)
                s = s + bias_ref[pl.ds(b * QBAND, QBAND), pl.ds(0, cols)]
                m = jnp.max(s, axis=1, keepdims=True)
                w = jnp.exp(s - m)
                den = jnp.sum(w, axis=1, keepdims=True, dtype=jnp.float32)
                ctx = jnp.dot(w,
                              vbuf[slot, hl, pl.ds(0, cols)].astype(
                                  jnp.bfloat16),
                              preferred_element_type=jnp.float32)
                ctx = ctx * (1.0 / den)
                ctx_buf[pl.ds(b * QBAND, QBAND), pl.ds(hl * DH, DH)] = (
                    ctx.astype(jnp.bfloat16))
            return carry

        lax.fori_loop(0, ncols // DH, head, 0)

        part = jnp.dot(ctx_buf[:, pl.ds(col0, ncols)], wo_c,
                       preferred_element_type=jnp.float32)
        if first:
            out_ref[0] = part
        else:
            out_ref[0] = out_ref[0] + part

    for d in loads[0]:
        d.wait()
    for d in loads[1]:
        d.start()
    process(0, wq_ref[...], wo_ref[...], 0, DM, first=True)

    for d in loads[1]:
        d.wait()
    for d in loads[2]:
        d.start()
    a_sends[0].wait_recv()
    a_sends[1].wait_recv()
    process(1, comm_ref[0, 0, :, pl.ds(0, HALF)],
            comm_ref[0, 1, pl.ds(0, HALF)], 0, HALF, first=False)
    a_sends[2].wait_recv()
    a_sends[3].wait_recv()
    b1.start()
    b2.start()
    for k in (4, 5, 6, 7):
        a_sends[k].wait_recv()
    b3.start()
    b4.start()
    process(1, comm_ref[0, 0, :, pl.ds(HALF, HALF)],
            comm_ref[0, 1, pl.ds(HALF, HALF)], HALF, HALF, first=False)

    for d in loads[2]:
        d.wait()
    for d in loads[3]:
        d.start()
    process(0, comm_ref[1, 0], comm_ref[1, 1], 0, DM, first=False)

    b1.wait_recv()
    b2.wait_recv()
    b3.wait_recv()
    b4.wait_recv()
    for d in loads[3]:
        d.wait()
    process(1, comm_ref[2, 0], comm_ref[2, 1], 0, DM, first=False)

    for r in a_sends + [b1, b2, b3, b4]:
        r.wait_send()


def kernel(x, Wq, K_ext, V_ext, Wo):
    my = lax.axis_index("i")

    x2 = (x[0] * SCALE).astype(jnp.bfloat16)

    out = pl.pallas_call(
        _body,
        out_shape=jax.ShapeDtypeStruct((1, SQ, DM), jnp.float32),
        in_specs=[
            pl.BlockSpec(memory_space=pltpu.VMEM),
            pl.BlockSpec(memory_space=pltpu.VMEM),
            pl.BlockSpec(memory_space=pltpu.VMEM),
            pl.BlockSpec(memory_space=pl.ANY),
            pl.BlockSpec(memory_space=pl.ANY),
        ],
        out_specs=pl.BlockSpec(memory_space=pltpu.VMEM),
        scratch_shapes=[
            pltpu.VMEM((3, 2, DM, DM), jnp.bfloat16),
            pltpu.VMEM((2, HQ_LOCAL, SKV, DH), jnp.float32),
            pltpu.VMEM((2, HQ_LOCAL, SKV, DH), jnp.float32),
            pltpu.VMEM((SQ, DM), jnp.bfloat16),
            pltpu.VMEM((SQ, DM), jnp.bfloat16),
            pltpu.VMEM((SQ, SKV), jnp.bfloat16),
            pltpu.SemaphoreType.DMA((12,)),
            pltpu.SemaphoreType.DMA((12,)),
            pltpu.SemaphoreType.DMA((2, 2, HQ_LOCAL)),
        ],
        compiler_params=pltpu.CompilerParams(
            collective_id=0,
            vmem_limit_bytes=100 * 1024 * 1024,
        ),
    )(x2, Wq.astype(jnp.bfloat16), Wo.astype(jnp.bfloat16), K_ext, V_ext)
    return out


# device time: 116235 ns/iter; 3.9600x vs baseline; 1.0084x over previous
import jax
import jax.numpy as jnp
from jax import lax
from jax.experimental import pallas as pl
from jax.experimental.pallas import tpu as pltpu

N_DEV = 4
SQ = 1024
SKV = 1024
DM = 1024
HQ_LOCAL = 8
DH = 128
BLK = 64
QBAND = 256
N_BANDS = SQ // QBAND
HALF = DM // 2
SCALE = 0.08838834764831843


def _body(x_ref, wq_ref, wo_ref, k_hbm, v_hbm, out_ref,
          comm_ref, kbuf, vbuf, q_buf, ctx_buf, bias_ref,
          send_sems, recv_sems, load_sems):
    my = lax.axis_index("i")
    right = lax.rem(my + 1, N_DEV)
    left = lax.rem(my + N_DEV - 1, N_DEV)

    barrier_sem = pltpu.get_barrier_semaphore()
    for nbr in (left, right):
        pl.semaphore_signal(
            barrier_sem, inc=1,
            device_id=(nbr,), device_id_type=pl.DeviceIdType.MESH,
        )
    pl.semaphore_wait(barrier_sem, 2)

    def remote(src, dst, s, dev):
        return pltpu.make_async_remote_copy(
            src_ref=src, dst_ref=dst,
            send_sem=send_sems.at[s], recv_sem=recv_sems.at[s],
            device_id=(dev,), device_id_type=pl.DeviceIdType.MESH,
        )

    a_sends = []
    for d, dev in ((0, right), (1, left)):
        for g in range(2):
            cs = pl.ds(g * HALF, HALF)
            base = 4 * d + 2 * g
            a_sends.append(remote(wq_ref.at[:, cs],
                                  comm_ref.at[d, 0, :, cs], base, dev))
            a_sends.append(remote(wo_ref.at[cs],
                                  comm_ref.at[d, 1, cs], base + 1, dev))
    b1 = remote(comm_ref.at[0, 0, pl.ds(0, HALF)],
                comm_ref.at[2, 0, pl.ds(0, HALF)], 8, right)
    b2 = remote(comm_ref.at[0, 1, pl.ds(0, HALF)],
                comm_ref.at[2, 1, pl.ds(0, HALF)], 9, right)
    b3 = remote(comm_ref.at[1, 0, pl.ds(HALF, HALF)],
                comm_ref.at[2, 0, pl.ds(HALF, HALF)], 10, left)
    b4 = remote(comm_ref.at[1, 1, pl.ds(HALF, HALF)],
                comm_ref.at[2, 1, pl.ds(HALF, HALF)], 11, left)

    for r in a_sends:
        r.start()

    chunk_ids = [my,
                 lax.rem(my + N_DEV - 1, N_DEV),
                 lax.rem(my + 1, N_DEV),
                 lax.rem(my + 2, N_DEV)]

    def kv_load(a):
        slot = a % 2
        cps = []
        for hl in range(HQ_LOCAL):
            h = chunk_ids[a] * HQ_LOCAL + hl
            cps.append(pltpu.make_async_copy(
                k_hbm.at[my, :, h], kbuf.at[slot, hl],
                load_sems.at[slot, 0, hl]))
            cps.append(pltpu.make_async_copy(
                v_hbm.at[my, :, h], vbuf.at[slot, hl],
                load_sems.at[slot, 1, hl]))
        return cps

    loads = [kv_load(a) for a in range(N_DEV)]
    for d in loads[0]:
        d.start()

    rowb = lax.broadcasted_iota(jnp.int32, (SQ, SKV), 0) // BLK
    colb = lax.broadcasted_iota(jnp.int32, (SQ, SKV), 1) // BLK
    bias_ref[...] = jnp.where(colb <= rowb, 0.0, -1e9).astype(jnp.bfloat16)

    x_bf = (x_ref[0] * SCALE).astype(jnp.bfloat16)

    def process(slot, wq_c, wo_c, col0, ncols, first):
        qg = jnp.dot(x_bf, wq_c, preferred_element_type=jnp.float32)
        q_buf[:, pl.ds(col0, ncols)] = qg.astype(jnp.bfloat16)

        def head(i, carry):
            hl = col0 // DH + i
            for b in range(N_BANDS):
                cols = (b + 1) * QBAND
                q = q_buf[pl.ds(b * QBAND, QBAND), pl.ds(hl * DH, DH)]
                s = lax.dot_general(
                    q, kbuf[slot, hl, pl.ds(0, cols)].astype(jnp.bfloat16),
                    (((1,), (1,)), ((), ())),
                    preferred_element_type=jnp.float32,
                ).astype(jnp.bfloat16)
                s = s + bias_ref[pl.ds(b * QBAND, QBAND), pl.ds(0, cols)]
                m = jnp.max(s, axis=1, keepdims=True)
                w = jnp.exp(s - m)
                den = jnp.sum(w, axis=1, keepdims=True, dtype=jnp.float32)
                ctx = jnp.dot(w,
                              vbuf[slot, hl, pl.ds(0, cols)].astype(
                                  jnp.bfloat16),
                              preferred_element_type=jnp.float32)
                ctx = ctx * (1.0 / den)
                ctx_buf[pl.ds(b * QBAND, QBAND), pl.ds(hl * DH, DH)] = (
                    ctx.astype(jnp.bfloat16))
            return carry

        lax.fori_loop(0, ncols // DH, head, 0)

        part = jnp.dot(ctx_buf[:, pl.ds(col0, ncols)], wo_c,
                       preferred_element_type=jnp.float32)
        if first:
            out_ref[0] = part
        else:
            out_ref[0] = out_ref[0] + part

    for d in loads[0]:
        d.wait()
    for d in loads[1]:
        d.start()
    process(0, wq_ref[...], wo_ref[...], 0, DM, first=True)

    for d in loads[1]:
        d.wait()
    for d in loads[2]:
        d.start()
    a_sends[0].wait_recv()
    a_sends[1].wait_recv()
    process(1, comm_ref[0, 0, :, pl.ds(0, HALF)],
            comm_ref[0, 1, pl.ds(0, HALF)], 0, HALF, first=False)
    a_sends[2].wait_recv()
    a_sends[3].wait_recv()
    b1.start()
    b2.start()
    for k in (4, 5, 6, 7):
        a_sends[k].wait_recv()
    b3.start()
    b4.start()
    process(1, comm_ref[0, 0, :, pl.ds(HALF, HALF)],
            comm_ref[0, 1, pl.ds(HALF, HALF)], HALF, HALF, first=False)

    for d in loads[2]:
        d.wait()
    for d in loads[3]:
        d.start()
    process(0, comm_ref[1, 0], comm_ref[1, 1], 0, DM, first=False)

    b1.wait_recv()
    b2.wait_recv()
    b3.wait_recv()
    b4.wait_recv()
    for d in loads[3]:
        d.wait()
    process(1, comm_ref[2, 0], comm_ref[2, 1], 0, DM, first=False)

    for r in a_sends + [b1, b2, b3, b4]:
        r.wait_send()


def kernel(x, Wq, K_ext, V_ext, Wo):
    out = pl.pallas_call(
        _body,
        out_shape=jax.ShapeDtypeStruct((1, SQ, DM), jnp.float32),
        in_specs=[
            pl.BlockSpec(memory_space=pltpu.VMEM),
            pl.BlockSpec(memory_space=pltpu.VMEM),
            pl.BlockSpec(memory_space=pltpu.VMEM),
            pl.BlockSpec(memory_space=pl.ANY),
            pl.BlockSpec(memory_space=pl.ANY),
        ],
        out_specs=pl.BlockSpec(memory_space=pltpu.VMEM),
        scratch_shapes=[
            pltpu.VMEM((3, 2, DM, DM), jnp.bfloat16),
            pltpu.VMEM((2, HQ_LOCAL, SKV, DH), jnp.float32),
            pltpu.VMEM((2, HQ_LOCAL, SKV, DH), jnp.float32),
            pltpu.VMEM((SQ, DM), jnp.bfloat16),
            pltpu.VMEM((SQ, DM), jnp.bfloat16),
            pltpu.VMEM((SQ, SKV), jnp.bfloat16),
            pltpu.SemaphoreType.DMA((12,)),
            pltpu.SemaphoreType.DMA((12,)),
            pltpu.SemaphoreType.DMA((2, 2, HQ_LOCAL)),
        ],
        compiler_params=pltpu.CompilerParams(
            collective_id=0,
            vmem_limit_bytes=100 * 1024 * 1024,
        ),
    )(x, Wq.astype(jnp.bfloat16), Wo.astype(jnp.bfloat16), K_ext, V_ext)
    return out
